# Initial kernel scaffold; baseline (speedup 1.0000x reference)
import jax
import jax.numpy as jnp
from jax.experimental import pallas as pl


def kernel(seq_item, last_memory, all_memory, mask, item_matrix, Wr, Ur, Vr):
    raise NotImplementedError("write your pallas kernel here")


# speedup vs baseline: 1.6466x; 1.6466x over previous
import functools

import jax
import jax.numpy as jnp
from jax import lax
from jax.experimental import pallas as pl
from jax.experimental.pallas import tpu as pltpu
from jax.experimental.pallas import tpu_sc as plsc

B = 1024
L = 50
H = 128
V = 1000

NC = 2
NS = 16
NW = NC * NS
ROWS_PER_W = B // NW
GROUPS = ROWS_PER_W // 16


def _probs_body(last_ref, all_ref, mask_ref, wr_ref, ur_ref, vr_ref, out_ref):
    bb = all_ref.shape[0]
    lm = lax.dot_general(
        last_ref[...], wr_ref[...], (((1,), (1,)), ((), ())),
        preferred_element_type=jnp.float32)
    am = lax.dot_general(
        all_ref[...].reshape(bb * L, H), ur_ref[...],
        (((1,), (1,)), ((), ())),
        preferred_element_type=jnp.float32)
    mem = jnp.tanh(am.reshape(bb, L, H) + lm[:, None, :])
    scores = lax.dot_general(
        mem.reshape(bb * L, H), vr_ref[...], (((1,), (1,)), ((), ())),
        preferred_element_type=jnp.float32).reshape(bb, L)
    scores = jnp.where(mask_ref[...] != 0, -1000000000.0, scores)
    m = jnp.max(scores, axis=-1, keepdims=True)
    e = jnp.exp(scores - m)
    out_ref[...] = e / jnp.sum(e, axis=-1, keepdims=True)


def _tc_probs(last_memory, all_memory, mask_i32, Wr, Ur, Vr, bb=128):
    grid = (B // bb,)
    return pl.pallas_call(
        _probs_body,
        grid=grid,
        in_specs=[
            pl.BlockSpec((bb, H), lambda i: (i, 0)),
            pl.BlockSpec((bb, L, H), lambda i: (i, 0, 0)),
            pl.BlockSpec((bb, L), lambda i: (i, 0)),
            pl.BlockSpec((H, H), lambda i: (0, 0)),
            pl.BlockSpec((H, H), lambda i: (0, 0)),
            pl.BlockSpec((1, H), lambda i: (0, 0)),
        ],
        out_specs=pl.BlockSpec((bb, L), lambda i: (i, 0)),
        out_shape=jax.ShapeDtypeStruct((B, L), jnp.float32),
    )(last_memory, all_memory, mask_i32, Wr, Ur, Vr)


def _sc_scatter_body(probs_hbm, seq_hbm, out_hbm, probs_v, seq_v, acc_v):
    wid = lax.axis_index("s") * NC + lax.axis_index("c")
    in_base = wid * (ROWS_PER_W * L)
    out_base = wid * (ROWS_PER_W * V)

    pltpu.sync_copy(probs_hbm.at[pl.ds(in_base, ROWS_PER_W * L)], probs_v)
    pltpu.sync_copy(seq_hbm.at[pl.ds(in_base, ROWS_PER_W * L)], seq_v)

    zeros16 = jnp.zeros((16,), jnp.float32)

    def _zero(i, _):
        acc_v[pl.ds(i * 16, 16)] = zeros16
        return 0

    lax.fori_loop(0, (ROWS_PER_W * V) // 16, _zero, 0)

    lane = lax.iota(jnp.int32, 16)
    for g in range(GROUPS):
        row = lane + g * 16
        lin_base = row * L
        acc_base = row * V
        for l in range(L):
            col = plsc.load_gather(seq_v, [lin_base + l])
            val = plsc.load_gather(probs_v, [lin_base + l])
            plsc.addupdate_scatter(acc_v, [acc_base + col], val)

    pltpu.sync_copy(acc_v, out_hbm.at[pl.ds(out_base, ROWS_PER_W * V)])


@functools.cache
def _sc_scatter():
    return pl.kernel(
        _sc_scatter_body,
        out_type=jax.ShapeDtypeStruct((B * V,), jnp.float32),
        mesh=plsc.VectorSubcoreMesh(core_axis_name="c", subcore_axis_name="s",
                                    num_cores=NC, num_subcores=NS),
        compiler_params=pltpu.CompilerParams(needs_layout_passes=False),
        scratch_types=[
            pltpu.VMEM((ROWS_PER_W * L,), jnp.float32),
            pltpu.VMEM((ROWS_PER_W * L,), jnp.int32),
            pltpu.VMEM((ROWS_PER_W * V,), jnp.float32),
        ],
    )


def kernel(seq_item, last_memory, all_memory, mask, item_matrix, Wr, Ur, Vr):
    probs = _tc_probs(last_memory, all_memory, mask.astype(jnp.int32),
                      Wr, Ur, Vr)
    out_flat = _sc_scatter()(probs.reshape(B * L),
                             seq_item.astype(jnp.int32).reshape(B * L))
    return out_flat.reshape(B, V)
